# baseline (speedup 1.0000x reference)
import functools

import jax
import jax.numpy as jnp
from jax import lax
from jax.experimental import pallas as pl
from jax.experimental.pallas import tpu as pltpu
from jax.experimental.pallas import tpu_sc as plsc

N_TOK = 8192
D = 4096
N_IDX = 4096
NC = 2
NS = 16
NW = NC * NS
PER_W = N_IDX // NW
G = 8
N_CHUNK = PER_W // G
LANES = 16
D_VECS = D // LANES
COLS_W = D // NW

_mesh = plsc.VectorSubcoreMesh(core_axis_name="c", subcore_axis_name="s")
_params = pltpu.CompilerParams(needs_layout_passes=False)


@functools.partial(
    pl.kernel,
    mesh=_mesh,
    compiler_params=_params,
    out_type=[
        jax.ShapeDtypeStruct((NW, D), jnp.float32),
        jax.ShapeDtypeStruct((NW, LANES), jnp.float32),
    ],
    scratch_types=[
        pltpu.VMEM((PER_W,), jnp.int32),
        pltpu.VMEM((PER_W,), jnp.float32),
        pltpu.VMEM((G, D), jnp.float32),
        pltpu.VMEM((G, D), jnp.float32),
        pltpu.VMEM((G, D), jnp.float32),
        pltpu.VMEM((D,), jnp.float32),
        pltpu.VMEM((LANES,), jnp.float32),
        pltpu.SemaphoreType.DMA,
        pltpu.SemaphoreType.DMA,
        pltpu.SemaphoreType.DMA,
        pltpu.SemaphoreType.DMA,
    ],
)
def _partial_sums(tokens_hbm, s_hbm, idx_hbm, acc_out, den_out,
                  idx_v, ssel_v, rows0_v, rows1_v, rows2_v, acc_v, den_v,
                  sem0, sem1, sem2, sems):
    wid = lax.axis_index("s") * NC + lax.axis_index("c")
    base = wid * PER_W

    pltpu.sync_copy(idx_hbm.at[pl.ds(base, PER_W)], idx_v)

    def gather_start(k, buf, sem):
        pltpu.async_copy(tokens_hbm.at[idx_v.at[pl.ds(k * G, G)]], buf, sem)

    def gather_wait(k, buf, sem):
        pltpu.make_async_copy(tokens_hbm.at[idx_v.at[pl.ds(k * G, G)]],
                              buf, sem).wait()

    gather_start(0, rows0_v, sem0)
    gather_start(1, rows1_v, sem1)
    gather_start(2, rows2_v, sem2)
    pltpu.async_copy(s_hbm.at[idx_v], ssel_v, sems).wait()

    den = jnp.zeros((LANES,), jnp.float32)
    for t in range(PER_W // LANES):
        den = den + ssel_v[pl.ds(t * LANES, LANES)]
    den_v[...] = den

    def process(buf, k, init=False):
        w = [plsc.load_gather(ssel_v, [jnp.full((LANES,), k * G + r, jnp.int32)])
             for r in range(G)]

        def col_body(j, _):
            sl = pl.ds(j * LANES, LANES)
            a = w[0] * buf[0, sl] if init else acc_v[sl] + w[0] * buf[0, sl]
            for r in range(1, G):
                a = a + w[r] * buf[r, sl]
            acc_v[sl] = a
            return 0
        lax.fori_loop(0, D_VECS, col_body, 0, unroll=4)

    gather_wait(0, rows0_v, sem0)
    process(rows0_v, 0, init=True)
    gather_start(3, rows0_v, sem0)

    bufs = (rows0_v, rows1_v, rows2_v)
    ring_sems = (sem0, sem1, sem2)

    def body(t, _):
        k = 3 * t
        for b in range(3):
            buf, sem = bufs[(b + 1) % 3], ring_sems[(b + 1) % 3]
            gather_wait(k + b + 1, buf, sem)
            process(buf, k + b + 1)

            @pl.when(k + b + 4 < N_CHUNK)
            def _():
                gather_start(k + b + 4, buf, sem)
        return 0
    lax.fori_loop(0, (N_CHUNK - 1) // 3, body, 0)

    pltpu.sync_copy(acc_v, acc_out.at[wid])
    pltpu.sync_copy(den_v, den_out.at[wid])


def _combine_tc(acc_ref, den_ref, out_ref):
    den = jnp.sum(den_ref[...]) + 1e-6
    out_ref[...] = jnp.sum(acc_ref[...], axis=0, keepdims=True) / den


def kernel(tokens, s, idx):
    idx32 = idx.astype(jnp.int32)
    acc, den = _partial_sums(tokens, s, idx32)
    return pl.pallas_call(
        _combine_tc,
        out_shape=jax.ShapeDtypeStruct((1, D), jnp.float32),
    )(acc, den)


# speedup vs baseline: 1.0382x; 1.0382x over previous
import functools

import jax
import jax.numpy as jnp
from jax import lax
from jax.experimental import pallas as pl
from jax.experimental.pallas import tpu as pltpu
from jax.experimental.pallas import tpu_sc as plsc

N_TOK = 8192
D = 4096
N_IDX = 4096
NC = 2
NS = 16
NW = NC * NS
PER_W = N_IDX // NW
G = 8
N_CHUNK = PER_W // G
LANES = 16
D_VECS = D // LANES
COLS_W = D // NW

_mesh = plsc.VectorSubcoreMesh(core_axis_name="c", subcore_axis_name="s")
_params = pltpu.CompilerParams(needs_layout_passes=False)


@functools.partial(
    pl.kernel,
    mesh=_mesh,
    compiler_params=_params,
    out_type=[
        jax.ShapeDtypeStruct((NW, D), jnp.float32),
        jax.ShapeDtypeStruct((NW, LANES), jnp.float32),
    ],
    scratch_types=[
        pltpu.VMEM((PER_W,), jnp.int32),
        pltpu.VMEM((PER_W,), jnp.float32),
        pltpu.VMEM((G, D), jnp.float32),
        pltpu.VMEM((G, D), jnp.float32),
        pltpu.VMEM((D,), jnp.float32),
        pltpu.VMEM((LANES,), jnp.float32),
        pltpu.SemaphoreType.DMA,
        pltpu.SemaphoreType.DMA,
        pltpu.SemaphoreType.DMA,
    ],
)
def _partial_sums(tokens_hbm, s_hbm, idx_hbm, acc_out, den_out,
                  idx_v, ssel_v, rows0_v, rows1_v, acc_v, den_v,
                  sem0, sem1, sems):
    wid = lax.axis_index("s") * NC + lax.axis_index("c")
    base = wid * PER_W

    pltpu.sync_copy(idx_hbm.at[pl.ds(base, PER_W)], idx_v)

    def gather_start(k, buf, sem):
        pltpu.async_copy(tokens_hbm.at[idx_v.at[pl.ds(k * G, G)]], buf, sem)

    def gather_wait(k, buf, sem):
        pltpu.make_async_copy(tokens_hbm.at[idx_v.at[pl.ds(k * G, G)]],
                              buf, sem).wait()

    gather_start(0, rows0_v, sem0)
    pltpu.async_copy(s_hbm.at[idx_v], ssel_v, sems).wait()

    den = jnp.zeros((LANES,), jnp.float32)
    for t in range(PER_W // LANES):
        den = den + ssel_v[pl.ds(t * LANES, LANES)]
    den_v[...] = den

    def process(buf, k, init=False):
        w = [plsc.load_gather(ssel_v, [jnp.full((LANES,), k * G + r, jnp.int32)])
             for r in range(G)]

        def col_body(j, _):
            sl = pl.ds(j * LANES, LANES)
            a = w[0] * buf[0, sl] if init else acc_v[sl] + w[0] * buf[0, sl]
            for r in range(1, G):
                a = a + w[r] * buf[r, sl]
            acc_v[sl] = a
            return 0
        lax.fori_loop(0, D_VECS, col_body, 0, unroll=4)

    gather_start(1, rows1_v, sem1)
    gather_wait(0, rows0_v, sem0)
    process(rows0_v, 0, init=True)

    def body(t, _):
        k0 = 2 * t
        gather_start(k0 + 2, rows0_v, sem0)
        gather_wait(k0 + 1, rows1_v, sem1)
        process(rows1_v, k0 + 1)
        gather_start(k0 + 3, rows1_v, sem1)
        gather_wait(k0 + 2, rows0_v, sem0)
        process(rows0_v, k0 + 2)
        return 0
    lax.fori_loop(0, N_CHUNK // 2 - 1, body, 0)

    gather_wait(N_CHUNK - 1, rows1_v, sem1)
    process(rows1_v, N_CHUNK - 1)

    pltpu.sync_copy(acc_v, acc_out.at[wid])
    pltpu.sync_copy(den_v, den_out.at[wid])


def _combine_tc(acc_ref, den_ref, out_ref):
    den = jnp.sum(den_ref[...]) + 1e-6
    out_ref[...] = jnp.sum(acc_ref[...], axis=0, keepdims=True) / den


def kernel(tokens, s, idx):
    idx32 = idx.astype(jnp.int32)
    acc, den = _partial_sums(tokens, s, idx32)
    return pl.pallas_call(
        _combine_tc,
        out_shape=jax.ShapeDtypeStruct((1, D), jnp.float32),
    )(acc, den)
